# baseline (speedup 1.0000x reference)
import functools

import jax
import jax.numpy as jnp
from jax import lax
from jax.experimental import pallas as pl
from jax.experimental.pallas import tpu as pltpu
from jax.experimental.pallas import tpu_sc as plsc

N = 10000
E = 320000
NP = 10240
NW = 32
CHUNK = 128
CPT = 80
EPAD = NW * CPT * CHUNK
RPT = NP // 16
F1 = 16
F2 = 48
K_BY_F = {F1: 8, F2: 5}
BLK = 1000

_MESH = plsc.VectorSubcoreMesh(core_axis_name="c", subcore_axis_name="s")
_SC_PARAMS = pltpu.CompilerParams(use_tc_tiling_on_sc=False)


def _zero_rows(vbuf, nrows, ncols):
    zero = jnp.zeros((16,), jnp.float32)

    def body(i, carry):
        for k in range(ncols // 16):
            vbuf[i, pl.ds(16 * k, 16)] = zero
        return carry

    lax.fori_loop(0, nrows, body, 0)


def _hist_body(dst_hbm, out_hbm, dst_v, ones_v, vbuf, acc_sh, sem):
    c = lax.axis_index("c")
    s = lax.axis_index("s")
    w = s * 2 + c
    pltpu.sync_copy(dst_hbm.at[w], dst_v)
    onehot = jnp.where(lax.iota(jnp.int32, 16) == 0, 1.0, 0.0)

    def obody(i, carry):
        ones_v[i, :] = onehot
        return carry

    lax.fori_loop(0, CHUNK, obody, 0)
    _zero_rows(vbuf, RPT, F1)
    pltpu.sync_copy(vbuf, acc_sh.at[pl.ds(s * RPT, RPT)])
    plsc.subcore_barrier()

    def gbody(g, carry):
        for k in range(16):
            pltpu.async_copy(ones_v, acc_sh.at[dst_v.at[g * 16 + k]], sem,
                             add=True)
        for k in range(16):
            pltpu.make_async_copy(ones_v, acc_sh.at[dst_v.at[g * 16 + k]],
                                  sem).wait()
        return carry

    lax.fori_loop(0, CPT // 16, gbody, 0)
    plsc.subcore_barrier()
    pltpu.sync_copy(acc_sh.at[pl.ds(s * RPT, RPT)], vbuf)
    pltpu.sync_copy(vbuf, out_hbm.at[c, pl.ds(s * RPT, RPT)])


_sc_hist = functools.partial(
    pl.kernel,
    mesh=_MESH,
    compiler_params=_SC_PARAMS,
    out_type=jax.ShapeDtypeStruct((2, NP, F1), jnp.float32),
    scratch_types=[
        pltpu.VMEM((CPT, CHUNK), jnp.int32),
        pltpu.VMEM((CHUNK, F1), jnp.float32),
        pltpu.VMEM((RPT, F1), jnp.float32),
        pltpu.VMEM_SHARED((NP, F1), jnp.float32),
        pltpu.SemaphoreType.DMA,
    ],
)(_hist_body)


def _scat_body(F, src_hbm, dst_hbm, tab_hbm, out_hbm, src_v, dst_v, rows_v,
               acc_sh, gsem, ssem):
    K = K_BY_F[F]
    G = CPT // K
    c = lax.axis_index("c")
    s = lax.axis_index("s")
    w = s * 2 + c
    pltpu.sync_copy(src_hbm.at[w], src_v)
    pltpu.sync_copy(dst_hbm.at[w], dst_v)
    zero = jnp.zeros((16,), jnp.float32)

    def zrow(i, carry):
        for t in range(F // 16):
            rows_v[0, i, pl.ds(16 * t, 16)] = zero
        return carry

    lax.fori_loop(0, CHUNK, zrow, 0)
    for i in range(RPT // CHUNK):
        pltpu.sync_copy(rows_v.at[0],
                        acc_sh.at[pl.ds(s * RPT + i * CHUNK, CHUNK)])
    plsc.subcore_barrier()

    def fire_gathers(g, buf):
        for k in range(K):
            pltpu.async_copy(tab_hbm.at[src_v.at[g * K + k]],
                             rows_v.at[buf + k], gsem)

    def drain_gathers(g, buf):
        for k in range(K):
            pltpu.make_async_copy(tab_hbm.at[src_v.at[g * K + k]],
                                  rows_v.at[buf + k], gsem).wait()

    def fire_scatters(g, buf):
        for k in range(K):
            pltpu.async_copy(rows_v.at[buf + k],
                             acc_sh.at[dst_v.at[g * K + k]], ssem, add=True)

    def drain_scatters(g, buf):
        for k in range(K):
            pltpu.make_async_copy(rows_v.at[buf + k],
                                  acc_sh.at[dst_v.at[g * K + k]], ssem).wait()

    fire_gathers(0, 0)

    def gbody(g, carry):
        buf = (g % 2) * K
        obuf = K - buf
        drain_gathers(g, buf)

        @pl.when(g >= 1)
        def _():
            drain_scatters(g - 1, obuf)

        @pl.when(g + 1 < G)
        def _():
            fire_gathers(g + 1, obuf)

        fire_scatters(g, buf)
        return carry

    lax.fori_loop(0, G, gbody, 0)
    drain_scatters(G - 1, ((G - 1) % 2) * K)
    plsc.subcore_barrier()
    for i in range(RPT // CHUNK):
        pltpu.sync_copy(acc_sh.at[pl.ds(s * RPT + i * CHUNK, CHUNK)],
                        rows_v.at[0])
        pltpu.sync_copy(rows_v.at[0],
                        out_hbm.at[c, pl.ds(s * RPT + i * CHUNK, CHUNK)])


def _make_scat(F):
    return functools.partial(
        pl.kernel,
        mesh=_MESH,
        compiler_params=_SC_PARAMS,
        out_type=jax.ShapeDtypeStruct((2, NP, F), jnp.float32),
        scratch_types=[
            pltpu.VMEM((CPT, CHUNK), jnp.int32),
            pltpu.VMEM((CPT, CHUNK), jnp.int32),
            pltpu.VMEM((2 * K_BY_F[F], CHUNK, F), jnp.float32),
            pltpu.VMEM_SHARED((NP, F), jnp.float32),
            pltpu.SemaphoreType.DMA,
            pltpu.SemaphoreType.DMA,
        ],
    )(functools.partial(_scat_body, F))


_sc_scat16 = _make_scat(F1)
_sc_scat48 = _make_scat(F2)


def _tc1_body(degp_ref, x_ref, w1_ref, hs_ref, dinv_ref):
    deg = degp_ref[0, :, 0:1] + degp_ref[1, :, 0:1] + 1.0
    dinv = lax.rsqrt(jnp.maximum(deg, 1e-12))
    h = jnp.dot(x_ref[...], w1_ref[...], preferred_element_type=jnp.float32)
    hs_ref[...] = h * dinv
    dinv_ref[...] = dinv


def _tc1(degp, x, W1):
    return pl.pallas_call(
        _tc1_body,
        grid=(N // BLK,),
        in_specs=[
            pl.BlockSpec((2, BLK, F1), lambda m: (0, m, 0)),
            pl.BlockSpec((BLK, 128), lambda m: (m, 0)),
            pl.BlockSpec((128, F1), lambda m: (0, 0)),
        ],
        out_specs=[
            pl.BlockSpec((BLK, F1), lambda m: (m, 0)),
            pl.BlockSpec((BLK, 1), lambda m: (m, 0)),
        ],
        out_shape=[
            jax.ShapeDtypeStruct((N, F1), jnp.float32),
            jax.ShapeDtypeStruct((N, 1), jnp.float32),
        ],
    )(degp, x, W1)


def _tc2_body(s1p_ref, hs1_ref, dinv_ref, b1_ref, w2p_ref, hs2_ref):
    dinv = dinv_ref[...]
    agg = dinv * (s1p_ref[0] + s1p_ref[1] + hs1_ref[...]) + b1_ref[...]
    g = jnp.maximum(agg, 0.0)
    h2 = jnp.dot(g, w2p_ref[...], preferred_element_type=jnp.float32)
    hs2_ref[...] = h2 * dinv


def _tc2(s1p, hs1, dinv, b1, W2p):
    return pl.pallas_call(
        _tc2_body,
        grid=(N // BLK,),
        in_specs=[
            pl.BlockSpec((2, BLK, F1), lambda m: (0, m, 0)),
            pl.BlockSpec((BLK, F1), lambda m: (m, 0)),
            pl.BlockSpec((BLK, 1), lambda m: (m, 0)),
            pl.BlockSpec((1, F1), lambda m: (0, 0)),
            pl.BlockSpec((F1, F2), lambda m: (0, 0)),
        ],
        out_specs=pl.BlockSpec((BLK, F2), lambda m: (m, 0)),
        out_shape=jax.ShapeDtypeStruct((N, F2), jnp.float32),
    )(s1p, hs1, dinv, b1, W2p)


def _tc3_body(s2p_ref, hs2_ref, dinv_ref, b2p_ref, out_ref):
    a = dinv_ref[...] * (s2p_ref[0] + s2p_ref[1] + hs2_ref[...]) + b2p_ref[...]
    a = a[:, :40]
    m = jnp.max(a, axis=1, keepdims=True)
    z = a - m
    lse = jnp.log(jnp.sum(jnp.exp(z), axis=1, keepdims=True))
    out_ref[...] = z - lse


def _tc3(s2p, hs2, dinv, b2p):
    return pl.pallas_call(
        _tc3_body,
        grid=(N // BLK,),
        in_specs=[
            pl.BlockSpec((2, BLK, F2), lambda m: (0, m, 0)),
            pl.BlockSpec((BLK, F2), lambda m: (m, 0)),
            pl.BlockSpec((BLK, 1), lambda m: (m, 0)),
            pl.BlockSpec((1, F2), lambda m: (0, 0)),
        ],
        out_specs=pl.BlockSpec((BLK, 40), lambda m: (m, 0)),
        out_shape=jax.ShapeDtypeStruct((N, 40), jnp.float32),
    )(s2p, hs2, dinv, b2p)


def kernel(x, edge_index, W1, b1, W2, b2):
    ei = edge_index.astype(jnp.int32)
    pad = EPAD - E
    src_p = jnp.concatenate([ei[0], jnp.zeros((pad,), jnp.int32)])
    dst_p = jnp.concatenate([ei[1], jnp.full((pad,), NP - 1, jnp.int32)])
    src_p = src_p.reshape(NW, CPT, CHUNK)
    dst_p = dst_p.reshape(NW, CPT, CHUNK)
    W2p = jnp.pad(W2, ((0, 0), (0, F2 - 40)))
    b2p = jnp.pad(b2, (0, F2 - 40)).reshape(1, F2)

    degp = _sc_hist(dst_p)
    hs1, dinv = _tc1(degp, x, W1)
    s1p = _sc_scat16(src_p, dst_p, hs1)
    hs2 = _tc2(s1p, hs1, dinv, b1.reshape(1, F1), W2p)
    s2p = _sc_scat48(src_p, dst_p, hs2)
    return _tc3(s2p, hs2, dinv, b2p)


# speedup vs baseline: 1.1278x; 1.1278x over previous
import functools

import jax
import jax.numpy as jnp
from jax import lax
from jax.experimental import pallas as pl
from jax.experimental.pallas import tpu as pltpu
from jax.experimental.pallas import tpu_sc as plsc

N = 10000
E = 320000
NP = 10240
NW = 32
CHUNK = 128
CPT = 80
EPAD = NW * CPT * CHUNK
RPT = NP // 16
F1 = 16
F2 = 48
K_BY_F = {F1: 8, F2: 5}
SPMEM_TAB_BY_F = {F1: True, F2: False}
BLK = 1000

_MESH = plsc.VectorSubcoreMesh(core_axis_name="c", subcore_axis_name="s")
_SC_PARAMS = pltpu.CompilerParams(use_tc_tiling_on_sc=False)


def _zero_rows(vbuf, nrows, ncols):
    zero = jnp.zeros((16,), jnp.float32)

    def body(i, carry):
        for k in range(ncols // 16):
            vbuf[i, pl.ds(16 * k, 16)] = zero
        return carry

    lax.fori_loop(0, nrows, body, 0)


def _hist_body(dst_hbm, out_hbm, dst_v, ones_v, vbuf, acc_sh, sem):
    c = lax.axis_index("c")
    s = lax.axis_index("s")
    w = s * 2 + c
    pltpu.sync_copy(dst_hbm.at[w], dst_v)
    onehot = jnp.where(lax.iota(jnp.int32, 16) == 0, 1.0, 0.0)

    def obody(i, carry):
        ones_v[i, :] = onehot
        return carry

    lax.fori_loop(0, CHUNK, obody, 0)
    _zero_rows(vbuf, RPT, F1)
    pltpu.sync_copy(vbuf, acc_sh.at[pl.ds(s * RPT, RPT)])
    plsc.subcore_barrier()

    def gbody(g, carry):
        for k in range(16):
            pltpu.async_copy(ones_v, acc_sh.at[dst_v.at[g * 16 + k]], sem,
                             add=True)
        for k in range(16):
            pltpu.make_async_copy(ones_v, acc_sh.at[dst_v.at[g * 16 + k]],
                                  sem).wait()
        return carry

    lax.fori_loop(0, CPT // 16, gbody, 0)
    plsc.subcore_barrier()
    pltpu.sync_copy(acc_sh.at[pl.ds(s * RPT, RPT)], vbuf)
    pltpu.sync_copy(vbuf, out_hbm.at[c, pl.ds(s * RPT, RPT)])


_sc_hist = functools.partial(
    pl.kernel,
    mesh=_MESH,
    compiler_params=_SC_PARAMS,
    out_type=jax.ShapeDtypeStruct((2, NP, F1), jnp.float32),
    scratch_types=[
        pltpu.VMEM((CPT, CHUNK), jnp.int32),
        pltpu.VMEM((CHUNK, F1), jnp.float32),
        pltpu.VMEM((RPT, F1), jnp.float32),
        pltpu.VMEM_SHARED((NP, F1), jnp.float32),
        pltpu.SemaphoreType.DMA,
    ],
)(_hist_body)


def _scat_body(F, src_hbm, dst_hbm, tab_hbm, out_hbm, src_v, dst_v, rows_v,
               acc_sh, gsem, ssem, *maybe_tab_sh):
    K = K_BY_F[F]
    G = CPT // K
    c = lax.axis_index("c")
    s = lax.axis_index("s")
    w = s * 2 + c
    pltpu.sync_copy(src_hbm.at[w], src_v)
    pltpu.sync_copy(dst_hbm.at[w], dst_v)
    if maybe_tab_sh:
        tab_src = maybe_tab_sh[0]
        pltpu.sync_copy(tab_hbm.at[pl.ds(s * RPT, RPT)],
                        tab_src.at[pl.ds(s * RPT, RPT)])
    else:
        tab_src = tab_hbm
    zero = jnp.zeros((16,), jnp.float32)

    def zrow(i, carry):
        for t in range(F // 16):
            rows_v[0, i, pl.ds(16 * t, 16)] = zero
        return carry

    lax.fori_loop(0, CHUNK, zrow, 0)
    for i in range(RPT // CHUNK):
        pltpu.sync_copy(rows_v.at[0],
                        acc_sh.at[pl.ds(s * RPT + i * CHUNK, CHUNK)])
    plsc.subcore_barrier()

    def fire_gathers(g, buf):
        for k in range(K):
            pltpu.async_copy(tab_src.at[src_v.at[g * K + k]],
                             rows_v.at[buf + k], gsem)

    def drain_gathers(g, buf):
        for k in range(K):
            pltpu.make_async_copy(tab_src.at[src_v.at[g * K + k]],
                                  rows_v.at[buf + k], gsem).wait()

    def fire_scatters(g, buf):
        for k in range(K):
            pltpu.async_copy(rows_v.at[buf + k],
                             acc_sh.at[dst_v.at[g * K + k]], ssem, add=True)

    def drain_scatters(g, buf):
        for k in range(K):
            pltpu.make_async_copy(rows_v.at[buf + k],
                                  acc_sh.at[dst_v.at[g * K + k]], ssem).wait()

    fire_gathers(0, 0)

    def gbody(g, carry):
        buf = (g % 2) * K
        obuf = K - buf
        drain_gathers(g, buf)

        @pl.when(g >= 1)
        def _():
            drain_scatters(g - 1, obuf)

        @pl.when(g + 1 < G)
        def _():
            fire_gathers(g + 1, obuf)

        fire_scatters(g, buf)
        return carry

    lax.fori_loop(0, G, gbody, 0)
    drain_scatters(G - 1, ((G - 1) % 2) * K)
    plsc.subcore_barrier()
    for i in range(RPT // CHUNK):
        pltpu.sync_copy(acc_sh.at[pl.ds(s * RPT + i * CHUNK, CHUNK)],
                        rows_v.at[0])
        pltpu.sync_copy(rows_v.at[0],
                        out_hbm.at[c, pl.ds(s * RPT + i * CHUNK, CHUNK)])


def _make_scat(F):
    scratch = [
        pltpu.VMEM((CPT, CHUNK), jnp.int32),
        pltpu.VMEM((CPT, CHUNK), jnp.int32),
        pltpu.VMEM((2 * K_BY_F[F], CHUNK, F), jnp.float32),
        pltpu.VMEM_SHARED((NP, F), jnp.float32),
        pltpu.SemaphoreType.DMA,
        pltpu.SemaphoreType.DMA,
    ]
    if SPMEM_TAB_BY_F[F]:
        scratch.append(pltpu.VMEM_SHARED((NP, F), jnp.float32))
    return functools.partial(
        pl.kernel,
        mesh=_MESH,
        compiler_params=_SC_PARAMS,
        out_type=jax.ShapeDtypeStruct((2, NP, F), jnp.float32),
        scratch_types=scratch,
    )(functools.partial(_scat_body, F))


_sc_scat16 = _make_scat(F1)
_sc_scat48 = _make_scat(F2)


def _tc1_body(degp_ref, x_ref, w1_ref, hs_ref, dinv_ref):
    deg = degp_ref[0, :, 0:1] + degp_ref[1, :, 0:1] + 1.0
    dinv = lax.rsqrt(jnp.maximum(deg, 1e-12))
    h = jnp.dot(x_ref[...], w1_ref[...], preferred_element_type=jnp.float32)
    hs_ref[...] = h * dinv
    dinv_ref[...] = dinv


def _tc1(degp, x, W1):
    return pl.pallas_call(
        _tc1_body,
        grid=(N // BLK,),
        in_specs=[
            pl.BlockSpec((2, BLK, F1), lambda m: (0, m, 0)),
            pl.BlockSpec((BLK, 128), lambda m: (m, 0)),
            pl.BlockSpec((128, F1), lambda m: (0, 0)),
        ],
        out_specs=[
            pl.BlockSpec((BLK, F1), lambda m: (m, 0)),
            pl.BlockSpec((BLK, 1), lambda m: (m, 0)),
        ],
        out_shape=[
            jax.ShapeDtypeStruct((N, F1), jnp.float32),
            jax.ShapeDtypeStruct((N, 1), jnp.float32),
        ],
    )(degp, x, W1)


def _tc2_body(s1p_ref, hs1_ref, dinv_ref, b1_ref, w2p_ref, hs2_ref):
    dinv = dinv_ref[...]
    agg = dinv * (s1p_ref[0] + s1p_ref[1] + hs1_ref[...]) + b1_ref[...]
    g = jnp.maximum(agg, 0.0)
    h2 = jnp.dot(g, w2p_ref[...], preferred_element_type=jnp.float32)
    hs2_ref[...] = h2 * dinv


def _tc2(s1p, hs1, dinv, b1, W2p):
    return pl.pallas_call(
        _tc2_body,
        grid=(N // BLK,),
        in_specs=[
            pl.BlockSpec((2, BLK, F1), lambda m: (0, m, 0)),
            pl.BlockSpec((BLK, F1), lambda m: (m, 0)),
            pl.BlockSpec((BLK, 1), lambda m: (m, 0)),
            pl.BlockSpec((1, F1), lambda m: (0, 0)),
            pl.BlockSpec((F1, F2), lambda m: (0, 0)),
        ],
        out_specs=pl.BlockSpec((BLK, F2), lambda m: (m, 0)),
        out_shape=jax.ShapeDtypeStruct((N, F2), jnp.float32),
    )(s1p, hs1, dinv, b1, W2p)


def _tc3_body(s2p_ref, hs2_ref, dinv_ref, b2p_ref, out_ref):
    a = dinv_ref[...] * (s2p_ref[0] + s2p_ref[1] + hs2_ref[...]) + b2p_ref[...]
    a = a[:, :40]
    m = jnp.max(a, axis=1, keepdims=True)
    z = a - m
    lse = jnp.log(jnp.sum(jnp.exp(z), axis=1, keepdims=True))
    out_ref[...] = z - lse


def _tc3(s2p, hs2, dinv, b2p):
    return pl.pallas_call(
        _tc3_body,
        grid=(N // BLK,),
        in_specs=[
            pl.BlockSpec((2, BLK, F2), lambda m: (0, m, 0)),
            pl.BlockSpec((BLK, F2), lambda m: (m, 0)),
            pl.BlockSpec((BLK, 1), lambda m: (m, 0)),
            pl.BlockSpec((1, F2), lambda m: (0, 0)),
        ],
        out_specs=pl.BlockSpec((BLK, 40), lambda m: (m, 0)),
        out_shape=jax.ShapeDtypeStruct((N, 40), jnp.float32),
    )(s2p, hs2, dinv, b2p)


def kernel(x, edge_index, W1, b1, W2, b2):
    ei = edge_index.astype(jnp.int32)
    pad = EPAD - E
    src_p = jnp.concatenate([ei[0], jnp.zeros((pad,), jnp.int32)])
    dst_p = jnp.concatenate([ei[1], jnp.full((pad,), NP - 1, jnp.int32)])
    src_p = src_p.reshape(NW, CPT, CHUNK)
    dst_p = dst_p.reshape(NW, CPT, CHUNK)
    W2p = jnp.pad(W2, ((0, 0), (0, F2 - 40)))
    b2p = jnp.pad(b2, (0, F2 - 40)).reshape(1, F2)

    degp = _sc_hist(dst_p)
    hs1, dinv = _tc1(degp, x, W1)
    s1p = _sc_scat16(src_p, dst_p, jnp.pad(hs1, ((0, NP - N), (0, 0))))
    hs2 = _tc2(s1p, hs1, dinv, b1.reshape(1, F1), W2p)
    s2p = _sc_scat48(src_p, dst_p, hs2)
    return _tc3(s2p, hs2, dinv, b2p)


# speedup vs baseline: 1.8478x; 1.6385x over previous
import functools

import jax
import jax.numpy as jnp
from jax import lax
from jax.experimental import pallas as pl
from jax.experimental.pallas import tpu as pltpu
from jax.experimental.pallas import tpu_sc as plsc

N = 10000
E = 320000
NP = 10240
NW = 32
CHUNK = 128
CPT = 80
EPAD = NW * CPT * CHUNK
RPT = NP // 16
F1 = 16
F2 = 48
K_BY_F = {F1: 8, F2: 2}
SPMEM_TAB_BY_F = {F1: True, F2: True}
BLK = 1000

_MESH = plsc.VectorSubcoreMesh(core_axis_name="c", subcore_axis_name="s")
_SC_PARAMS = pltpu.CompilerParams(use_tc_tiling_on_sc=False)


def _zero_rows(vbuf, nrows, ncols):
    zero = jnp.zeros((16,), jnp.float32)

    def body(i, carry):
        for k in range(ncols // 16):
            vbuf[i, pl.ds(16 * k, 16)] = zero
        return carry

    lax.fori_loop(0, nrows, body, 0)


def _hist_body(dst_hbm, out_hbm, dst_v, ones_v, vbuf, acc_sh, sem):
    c = lax.axis_index("c")
    s = lax.axis_index("s")
    w = s * 2 + c
    pltpu.sync_copy(dst_hbm.at[w], dst_v)
    onehot = jnp.where(lax.iota(jnp.int32, 16) == 0, 1.0, 0.0)

    def obody(i, carry):
        ones_v[i, :] = onehot
        return carry

    lax.fori_loop(0, CHUNK, obody, 0)
    _zero_rows(vbuf, RPT, F1)
    pltpu.sync_copy(vbuf, acc_sh.at[pl.ds(s * RPT, RPT)])
    plsc.subcore_barrier()

    def gbody(g, carry):
        for k in range(16):
            pltpu.async_copy(ones_v, acc_sh.at[dst_v.at[g * 16 + k]], sem,
                             add=True)
        for k in range(16):
            pltpu.make_async_copy(ones_v, acc_sh.at[dst_v.at[g * 16 + k]],
                                  sem).wait()
        return carry

    lax.fori_loop(0, CPT // 16, gbody, 0)
    plsc.subcore_barrier()
    pltpu.sync_copy(acc_sh.at[pl.ds(s * RPT, RPT)], vbuf)
    pltpu.sync_copy(vbuf, out_hbm.at[c, pl.ds(s * RPT, RPT)])


_sc_hist = functools.partial(
    pl.kernel,
    mesh=_MESH,
    compiler_params=_SC_PARAMS,
    out_type=jax.ShapeDtypeStruct((2, NP, F1), jnp.float32),
    scratch_types=[
        pltpu.VMEM((CPT, CHUNK), jnp.int32),
        pltpu.VMEM((CHUNK, F1), jnp.float32),
        pltpu.VMEM((RPT, F1), jnp.float32),
        pltpu.VMEM_SHARED((NP, F1), jnp.float32),
        pltpu.SemaphoreType.DMA,
    ],
)(_hist_body)


def _scat_body(F, src_hbm, dst_hbm, tab_hbm, out_hbm, src_v, dst_v, rows_v,
               acc_sh, gsem, ssem, *maybe_tab_sh):
    K = K_BY_F[F]
    G = CPT // K
    c = lax.axis_index("c")
    s = lax.axis_index("s")
    w = s * 2 + c
    pltpu.sync_copy(src_hbm.at[w], src_v)
    pltpu.sync_copy(dst_hbm.at[w], dst_v)
    if maybe_tab_sh:
        tab_src = maybe_tab_sh[0]
        pltpu.sync_copy(tab_hbm.at[pl.ds(s * RPT, RPT)],
                        tab_src.at[pl.ds(s * RPT, RPT)])
    else:
        tab_src = tab_hbm
    zero = jnp.zeros((16,), jnp.float32)

    def zrow(i, carry):
        for t in range(F // 16):
            rows_v[0, i, pl.ds(16 * t, 16)] = zero
        return carry

    lax.fori_loop(0, CHUNK, zrow, 0)
    for i in range(RPT // CHUNK):
        pltpu.sync_copy(rows_v.at[0],
                        acc_sh.at[pl.ds(s * RPT + i * CHUNK, CHUNK)])
    plsc.subcore_barrier()

    def fire_gathers(g, buf):
        for k in range(K):
            pltpu.async_copy(tab_src.at[src_v.at[g * K + k]],
                             rows_v.at[buf + k], gsem)

    def drain_gathers(g, buf):
        for k in range(K):
            pltpu.make_async_copy(tab_src.at[src_v.at[g * K + k]],
                                  rows_v.at[buf + k], gsem).wait()

    def fire_scatters(g, buf):
        for k in range(K):
            pltpu.async_copy(rows_v.at[buf + k],
                             acc_sh.at[dst_v.at[g * K + k]], ssem, add=True)

    def drain_scatters(g, buf):
        for k in range(K):
            pltpu.make_async_copy(rows_v.at[buf + k],
                                  acc_sh.at[dst_v.at[g * K + k]], ssem).wait()

    fire_gathers(0, 0)

    def gbody(g, carry):
        buf = (g % 2) * K
        obuf = K - buf
        drain_gathers(g, buf)

        @pl.when(g >= 1)
        def _():
            drain_scatters(g - 1, obuf)

        @pl.when(g + 1 < G)
        def _():
            fire_gathers(g + 1, obuf)

        fire_scatters(g, buf)
        return carry

    lax.fori_loop(0, G, gbody, 0)
    drain_scatters(G - 1, ((G - 1) % 2) * K)
    plsc.subcore_barrier()
    for i in range(RPT // CHUNK):
        pltpu.sync_copy(acc_sh.at[pl.ds(s * RPT + i * CHUNK, CHUNK)],
                        rows_v.at[0])
        pltpu.sync_copy(rows_v.at[0],
                        out_hbm.at[c, pl.ds(s * RPT + i * CHUNK, CHUNK)])


def _make_scat(F):
    scratch = [
        pltpu.VMEM((CPT, CHUNK), jnp.int32),
        pltpu.VMEM((CPT, CHUNK), jnp.int32),
        pltpu.VMEM((2 * K_BY_F[F], CHUNK, F), jnp.float32),
        pltpu.VMEM_SHARED((NP, F), jnp.float32),
        pltpu.SemaphoreType.DMA,
        pltpu.SemaphoreType.DMA,
    ]
    if SPMEM_TAB_BY_F[F]:
        scratch.append(pltpu.VMEM_SHARED((NP, F), jnp.float32))
    return functools.partial(
        pl.kernel,
        mesh=_MESH,
        compiler_params=_SC_PARAMS,
        out_type=jax.ShapeDtypeStruct((2, NP, F), jnp.float32),
        scratch_types=scratch,
    )(functools.partial(_scat_body, F))


_sc_scat16 = _make_scat(F1)
_sc_scat48 = _make_scat(F2)


def _tc1_body(degp_ref, x_ref, w1_ref, hs_ref, dinv_ref):
    deg = degp_ref[0, :, 0:1] + degp_ref[1, :, 0:1] + 1.0
    dinv = lax.rsqrt(jnp.maximum(deg, 1e-12))
    h = jnp.dot(x_ref[...], w1_ref[...], preferred_element_type=jnp.float32)
    hs_ref[...] = h * dinv
    dinv_ref[...] = dinv


def _tc1(degp, x, W1):
    return pl.pallas_call(
        _tc1_body,
        grid=(N // BLK,),
        in_specs=[
            pl.BlockSpec((2, BLK, F1), lambda m: (0, m, 0)),
            pl.BlockSpec((BLK, 128), lambda m: (m, 0)),
            pl.BlockSpec((128, F1), lambda m: (0, 0)),
        ],
        out_specs=[
            pl.BlockSpec((BLK, F1), lambda m: (m, 0)),
            pl.BlockSpec((BLK, 1), lambda m: (m, 0)),
        ],
        out_shape=[
            jax.ShapeDtypeStruct((N, F1), jnp.float32),
            jax.ShapeDtypeStruct((N, 1), jnp.float32),
        ],
    )(degp, x, W1)


def _tc2_body(s1p_ref, hs1_ref, dinv_ref, b1_ref, w2p_ref, hs2_ref):
    dinv = dinv_ref[...]
    agg = dinv * (s1p_ref[0] + s1p_ref[1] + hs1_ref[...]) + b1_ref[...]
    g = jnp.maximum(agg, 0.0)
    h2 = jnp.dot(g, w2p_ref[...], preferred_element_type=jnp.float32)
    hs2_ref[...] = h2 * dinv


def _tc2(s1p, hs1, dinv, b1, W2p):
    return pl.pallas_call(
        _tc2_body,
        grid=(N // BLK,),
        in_specs=[
            pl.BlockSpec((2, BLK, F1), lambda m: (0, m, 0)),
            pl.BlockSpec((BLK, F1), lambda m: (m, 0)),
            pl.BlockSpec((BLK, 1), lambda m: (m, 0)),
            pl.BlockSpec((1, F1), lambda m: (0, 0)),
            pl.BlockSpec((F1, F2), lambda m: (0, 0)),
        ],
        out_specs=pl.BlockSpec((BLK, F2), lambda m: (m, 0)),
        out_shape=jax.ShapeDtypeStruct((N, F2), jnp.float32),
    )(s1p, hs1, dinv, b1, W2p)


def _tc3_body(s2p_ref, hs2_ref, dinv_ref, b2p_ref, out_ref):
    a = dinv_ref[...] * (s2p_ref[0] + s2p_ref[1] + hs2_ref[...]) + b2p_ref[...]
    a = a[:, :40]
    m = jnp.max(a, axis=1, keepdims=True)
    z = a - m
    lse = jnp.log(jnp.sum(jnp.exp(z), axis=1, keepdims=True))
    out_ref[...] = z - lse


def _tc3(s2p, hs2, dinv, b2p):
    return pl.pallas_call(
        _tc3_body,
        grid=(N // BLK,),
        in_specs=[
            pl.BlockSpec((2, BLK, F2), lambda m: (0, m, 0)),
            pl.BlockSpec((BLK, F2), lambda m: (m, 0)),
            pl.BlockSpec((BLK, 1), lambda m: (m, 0)),
            pl.BlockSpec((1, F2), lambda m: (0, 0)),
        ],
        out_specs=pl.BlockSpec((BLK, 40), lambda m: (m, 0)),
        out_shape=jax.ShapeDtypeStruct((N, 40), jnp.float32),
    )(s2p, hs2, dinv, b2p)


def kernel(x, edge_index, W1, b1, W2, b2):
    ei = edge_index.astype(jnp.int32)
    pad = EPAD - E
    src_p = jnp.concatenate([ei[0], jnp.zeros((pad,), jnp.int32)])
    dst_p = jnp.concatenate([ei[1], jnp.full((pad,), NP - 1, jnp.int32)])
    src_p = src_p.reshape(NW, CPT, CHUNK)
    dst_p = dst_p.reshape(NW, CPT, CHUNK)
    W2p = jnp.pad(W2, ((0, 0), (0, F2 - 40)))
    b2p = jnp.pad(b2, (0, F2 - 40)).reshape(1, F2)

    degp = _sc_hist(dst_p)
    hs1, dinv = _tc1(degp, x, W1)
    s1p = _sc_scat16(src_p, dst_p, jnp.pad(hs1, ((0, NP - N), (0, 0))))
    hs2 = _tc2(s1p, hs1, dinv, b1.reshape(1, F1), W2p)
    s2p = _sc_scat48(src_p, dst_p, jnp.pad(hs2, ((0, NP - N), (0, 0))))
    return _tc3(s2p, hs2, dinv, b2p)


# speedup vs baseline: 1.8828x; 1.0189x over previous
import functools

import jax
import jax.numpy as jnp
from jax import lax
from jax.experimental import pallas as pl
from jax.experimental.pallas import tpu as pltpu
from jax.experimental.pallas import tpu_sc as plsc

N = 10000
E = 320000
NP = 10240
NW = 32
CHUNK = 128
CPT = 80
EPAD = NW * CPT * CHUNK
RPT = NP // 16
F1 = 16
F2 = 48
K_BY_F = {F1: 8, F2: 2}
SPMEM_TAB_BY_F = {F1: True, F2: True}
BLK = 1000

_MESH = plsc.VectorSubcoreMesh(core_axis_name="c", subcore_axis_name="s")
_SC_PARAMS = pltpu.CompilerParams(use_tc_tiling_on_sc=False)


def _zero_rows(vbuf, nrows, ncols):
    zero = jnp.zeros((16,), jnp.float32)

    def body(i, carry):
        for k in range(ncols // 16):
            vbuf[i, pl.ds(16 * k, 16)] = zero
        return carry

    lax.fori_loop(0, nrows, body, 0)


def _hist_body(dst_hbm, out_hbm, dst_v, ones_v, vbuf, acc_sh, sem):
    c = lax.axis_index("c")
    s = lax.axis_index("s")
    w = s * 2 + c
    pltpu.sync_copy(dst_hbm.at[w], dst_v)
    one = jnp.ones((16,), jnp.float32)

    def obody(i, carry):
        ones_v[pl.ds(i * 16, 16)] = one
        return carry

    lax.fori_loop(0, CHUNK // 16, obody, 0)
    zero = jnp.zeros((16,), jnp.float32)

    def zbody(i, carry):
        vbuf[pl.ds(i * 16, 16)] = zero
        return carry

    lax.fori_loop(0, RPT // 16, zbody, 0)
    pltpu.sync_copy(vbuf, acc_sh.at[pl.ds(s * RPT, RPT)])
    plsc.subcore_barrier()

    def gbody(g, carry):
        for k in range(16):
            pltpu.async_copy(ones_v, acc_sh.at[dst_v.at[g * 16 + k]], sem,
                             add=True)
        for k in range(16):
            pltpu.make_async_copy(ones_v, acc_sh.at[dst_v.at[g * 16 + k]],
                                  sem).wait()
        return carry

    lax.fori_loop(0, CPT // 16, gbody, 0)
    plsc.subcore_barrier()
    pltpu.sync_copy(acc_sh.at[pl.ds(s * RPT, RPT)], vbuf)
    pltpu.sync_copy(vbuf, out_hbm.at[c, pl.ds(s * RPT, RPT)])


_sc_hist = functools.partial(
    pl.kernel,
    mesh=_MESH,
    compiler_params=_SC_PARAMS,
    out_type=jax.ShapeDtypeStruct((2, NP), jnp.float32),
    scratch_types=[
        pltpu.VMEM((CPT, CHUNK), jnp.int32),
        pltpu.VMEM((CHUNK,), jnp.float32),
        pltpu.VMEM((RPT,), jnp.float32),
        pltpu.VMEM_SHARED((NP,), jnp.float32),
        pltpu.SemaphoreType.DMA,
    ],
)(_hist_body)


def _scat_body(F, src_hbm, dst_hbm, tab_hbm, out_hbm, src_v, dst_v, rows_v,
               acc_sh, gsem, ssem, *maybe_tab_sh):
    K = K_BY_F[F]
    G = CPT // K
    c = lax.axis_index("c")
    s = lax.axis_index("s")
    w = s * 2 + c
    pltpu.sync_copy(src_hbm.at[w], src_v)
    pltpu.sync_copy(dst_hbm.at[w], dst_v)
    if maybe_tab_sh:
        tab_src = maybe_tab_sh[0]
        pltpu.sync_copy(tab_hbm.at[pl.ds(s * RPT, RPT)],
                        tab_src.at[pl.ds(s * RPT, RPT)])
    else:
        tab_src = tab_hbm
    zero = jnp.zeros((16,), jnp.float32)

    def zrow(i, carry):
        for t in range(F // 16):
            rows_v[0, i, pl.ds(16 * t, 16)] = zero
        return carry

    lax.fori_loop(0, CHUNK, zrow, 0)
    for i in range(RPT // CHUNK):
        pltpu.sync_copy(rows_v.at[0],
                        acc_sh.at[pl.ds(s * RPT + i * CHUNK, CHUNK)])
    plsc.subcore_barrier()

    def fire_gathers(g, buf):
        for k in range(K):
            pltpu.async_copy(tab_src.at[src_v.at[g * K + k]],
                             rows_v.at[buf + k], gsem)

    def drain_gathers(g, buf):
        for k in range(K):
            pltpu.make_async_copy(tab_src.at[src_v.at[g * K + k]],
                                  rows_v.at[buf + k], gsem).wait()

    def fire_scatters(g, buf):
        for k in range(K):
            pltpu.async_copy(rows_v.at[buf + k],
                             acc_sh.at[dst_v.at[g * K + k]], ssem, add=True)

    def drain_scatters(g, buf):
        for k in range(K):
            pltpu.make_async_copy(rows_v.at[buf + k],
                                  acc_sh.at[dst_v.at[g * K + k]], ssem).wait()

    fire_gathers(0, 0)

    def gbody(g, carry):
        buf = (g % 2) * K
        obuf = K - buf
        drain_gathers(g, buf)

        @pl.when(g >= 1)
        def _():
            drain_scatters(g - 1, obuf)

        @pl.when(g + 1 < G)
        def _():
            fire_gathers(g + 1, obuf)

        fire_scatters(g, buf)
        return carry

    lax.fori_loop(0, G, gbody, 0)
    drain_scatters(G - 1, ((G - 1) % 2) * K)
    plsc.subcore_barrier()
    for i in range(RPT // CHUNK):
        pltpu.sync_copy(acc_sh.at[pl.ds(s * RPT + i * CHUNK, CHUNK)],
                        rows_v.at[0])
        pltpu.sync_copy(rows_v.at[0],
                        out_hbm.at[c, pl.ds(s * RPT + i * CHUNK, CHUNK)])


def _make_scat(F):
    scratch = [
        pltpu.VMEM((CPT, CHUNK), jnp.int32),
        pltpu.VMEM((CPT, CHUNK), jnp.int32),
        pltpu.VMEM((2 * K_BY_F[F], CHUNK, F), jnp.float32),
        pltpu.VMEM_SHARED((NP, F), jnp.float32),
        pltpu.SemaphoreType.DMA,
        pltpu.SemaphoreType.DMA,
    ]
    if SPMEM_TAB_BY_F[F]:
        scratch.append(pltpu.VMEM_SHARED((NP, F), jnp.float32))
    return functools.partial(
        pl.kernel,
        mesh=_MESH,
        compiler_params=_SC_PARAMS,
        out_type=jax.ShapeDtypeStruct((2, NP, F), jnp.float32),
        scratch_types=scratch,
    )(functools.partial(_scat_body, F))


_sc_scat16 = _make_scat(F1)
_sc_scat48 = _make_scat(F2)


def _tc1_body(degp_ref, x_ref, w1_ref, hs_ref, dinv_ref):
    deg = degp_ref[0] + degp_ref[1] + 1.0
    dinv = lax.rsqrt(jnp.maximum(deg, 1e-12))
    h = jnp.dot(x_ref[...], w1_ref[...], preferred_element_type=jnp.float32)
    hs_ref[...] = h * dinv
    dinv_ref[...] = dinv


def _tc1(degp, x, W1):
    return pl.pallas_call(
        _tc1_body,
        grid=(N // BLK,),
        in_specs=[
            pl.BlockSpec((2, BLK, 1), lambda m: (0, m, 0)),
            pl.BlockSpec((BLK, 128), lambda m: (m, 0)),
            pl.BlockSpec((128, F1), lambda m: (0, 0)),
        ],
        out_specs=[
            pl.BlockSpec((BLK, F1), lambda m: (m, 0)),
            pl.BlockSpec((BLK, 1), lambda m: (m, 0)),
        ],
        out_shape=[
            jax.ShapeDtypeStruct((N, F1), jnp.float32),
            jax.ShapeDtypeStruct((N, 1), jnp.float32),
        ],
    )(degp, x, W1)


def _tc2_body(s1p_ref, hs1_ref, dinv_ref, b1_ref, w2p_ref, hs2_ref):
    dinv = dinv_ref[...]
    agg = dinv * (s1p_ref[0] + s1p_ref[1] + hs1_ref[...]) + b1_ref[...]
    g = jnp.maximum(agg, 0.0)
    h2 = jnp.dot(g, w2p_ref[...], preferred_element_type=jnp.float32)
    hs2_ref[...] = h2 * dinv


def _tc2(s1p, hs1, dinv, b1, W2p):
    return pl.pallas_call(
        _tc2_body,
        grid=(N // BLK,),
        in_specs=[
            pl.BlockSpec((2, BLK, F1), lambda m: (0, m, 0)),
            pl.BlockSpec((BLK, F1), lambda m: (m, 0)),
            pl.BlockSpec((BLK, 1), lambda m: (m, 0)),
            pl.BlockSpec((1, F1), lambda m: (0, 0)),
            pl.BlockSpec((F1, F2), lambda m: (0, 0)),
        ],
        out_specs=pl.BlockSpec((BLK, F2), lambda m: (m, 0)),
        out_shape=jax.ShapeDtypeStruct((N, F2), jnp.float32),
    )(s1p, hs1, dinv, b1, W2p)


def _tc3_body(s2p_ref, hs2_ref, dinv_ref, b2p_ref, out_ref):
    a = dinv_ref[...] * (s2p_ref[0] + s2p_ref[1] + hs2_ref[...]) + b2p_ref[...]
    a = a[:, :40]
    m = jnp.max(a, axis=1, keepdims=True)
    z = a - m
    lse = jnp.log(jnp.sum(jnp.exp(z), axis=1, keepdims=True))
    out_ref[...] = z - lse


def _tc3(s2p, hs2, dinv, b2p):
    return pl.pallas_call(
        _tc3_body,
        grid=(N // BLK,),
        in_specs=[
            pl.BlockSpec((2, BLK, F2), lambda m: (0, m, 0)),
            pl.BlockSpec((BLK, F2), lambda m: (m, 0)),
            pl.BlockSpec((BLK, 1), lambda m: (m, 0)),
            pl.BlockSpec((1, F2), lambda m: (0, 0)),
        ],
        out_specs=pl.BlockSpec((BLK, 40), lambda m: (m, 0)),
        out_shape=jax.ShapeDtypeStruct((N, 40), jnp.float32),
    )(s2p, hs2, dinv, b2p)


def kernel(x, edge_index, W1, b1, W2, b2):
    ei = edge_index.astype(jnp.int32)
    pad = EPAD - E
    src_p = jnp.concatenate([ei[0], jnp.zeros((pad,), jnp.int32)])
    dst_p = jnp.concatenate([ei[1], jnp.full((pad,), NP - 1, jnp.int32)])
    src_p = src_p.reshape(NW, CPT, CHUNK)
    dst_p = dst_p.reshape(NW, CPT, CHUNK)
    W2p = jnp.pad(W2, ((0, 0), (0, F2 - 40)))
    b2p = jnp.pad(b2, (0, F2 - 40)).reshape(1, F2)

    degp = _sc_hist(dst_p)[:, :, None]
    hs1, dinv = _tc1(degp, x, W1)
    s1p = _sc_scat16(src_p, dst_p, jnp.pad(hs1, ((0, NP - N), (0, 0))))
    hs2 = _tc2(s1p, hs1, dinv, b1.reshape(1, F1), W2p)
    s2p = _sc_scat48(src_p, dst_p, jnp.pad(hs2, ((0, NP - N), (0, 0))))
    return _tc3(s2p, hs2, dinv, b2p)


# speedup vs baseline: 1.8912x; 1.0045x over previous
import functools

import jax
import jax.numpy as jnp
from jax import lax
from jax.experimental import pallas as pl
from jax.experimental.pallas import tpu as pltpu
from jax.experimental.pallas import tpu_sc as plsc

N = 10000
E = 320000
NP = 10240
NW = 32
CHUNK = 128
CPT = 80
EPAD = NW * CPT * CHUNK
RPT = NP // 16
F1 = 16
F2 = 48
K_BY_F = {F1: 8, F2: 2}
SPMEM_TAB_BY_F = {F1: True, F2: True}
BLK = 1000

_MESH = plsc.VectorSubcoreMesh(core_axis_name="c", subcore_axis_name="s")
_SC_PARAMS = pltpu.CompilerParams(use_tc_tiling_on_sc=False,
                                  needs_layout_passes=False)


def _fused1_body(src_hbm, dst_hbm, tab_hbm, s1_out, dinv_out, src_v, dsta_v,
                 dstb_v, dsto_v, rows_v, ones_v, degb_v, dinv_v, tabb_v,
                 hist_sh, acc_sh, tab_sh, gsem, ssem):
    K = K_BY_F[F1]
    G = CPT // K
    c = lax.axis_index("c")
    s = lax.axis_index("s")
    w = s * 2 + c
    pltpu.sync_copy(src_hbm.at[w], src_v)
    pltpu.sync_copy(dst_hbm.at[w], dsto_v)
    pltpu.sync_copy(dst_hbm.at[s * 2], dsta_v)
    pltpu.sync_copy(dst_hbm.at[s * 2 + 1], dstb_v)
    pltpu.sync_copy(tab_hbm.at[pl.ds(s * RPT, RPT)],
                    tab_sh.at[pl.ds(s * RPT, RPT)])
    one = jnp.ones((16,), jnp.float32)
    zero = jnp.zeros((16,), jnp.float32)

    def obody(i, carry):
        ones_v[pl.ds(i * 16, 16)] = one
        return carry

    lax.fori_loop(0, CHUNK // 16, obody, 0)

    def zb(i, carry):
        degb_v[pl.ds(i * 16, 16)] = zero
        return carry

    lax.fori_loop(0, RPT // 16, zb, 0)
    pltpu.sync_copy(degb_v, hist_sh.at[pl.ds(s * RPT, RPT)])

    def zrow(i, carry):
        rows_v[0, i, :] = zero
        return carry

    lax.fori_loop(0, CHUNK, zrow, 0)
    for i in range(RPT // CHUNK):
        pltpu.sync_copy(rows_v.at[0],
                        acc_sh.at[pl.ds(s * RPT + i * CHUNK, CHUNK)])
    plsc.subcore_barrier()

    for dv in (dsta_v, dstb_v):
        def hb(g, carry, dv=dv):
            for k in range(16):
                pltpu.async_copy(ones_v, hist_sh.at[dv.at[g * 16 + k]], ssem,
                                 add=True)
            for k in range(16):
                pltpu.make_async_copy(ones_v, hist_sh.at[dv.at[g * 16 + k]],
                                      ssem).wait()
            return carry

        lax.fori_loop(0, CPT // 16, hb, 0)
    plsc.subcore_barrier()

    pltpu.sync_copy(hist_sh.at[pl.ds(s * RPT, RPT)], degb_v)
    half = jnp.full((16,), 0.5, jnp.float32)
    th = jnp.full((16,), 1.5, jnp.float32)

    def db(i, carry):
        d = degb_v[pl.ds(i * 16, 16)] + 1.0
        ii = plsc.bitcast(d, jnp.int32)
        ii = 0x5F3759DF - lax.shift_right_logical(ii, 1)
        y = plsc.bitcast(ii, jnp.float32)
        hx = d * half
        for _ in range(3):
            y = y * (th - hx * y * y)
        dinv_v[pl.ds(i * 16, 16)] = y
        return carry

    lax.fori_loop(0, RPT // 16, db, 0)
    pltpu.sync_copy(dinv_v, dinv_out.at[c, pl.ds(s * RPT, RPT)])
    for i in range(RPT // CHUNK):
        pltpu.sync_copy(tab_sh.at[pl.ds(s * RPT + i * CHUNK, CHUNK)], tabb_v)

        def sb(rb, carry, i=i):
            dv16 = dinv_v[pl.ds(i * CHUNK + rb * 16, 16)]
            for r in range(16):
                b = jnp.broadcast_to(dv16[r], (16,))
                tabb_v[rb * 16 + r, :] = tabb_v[rb * 16 + r, :] * b
            return carry

        lax.fori_loop(0, CHUNK // 16, sb, 0)
        pltpu.sync_copy(tabb_v, tab_sh.at[pl.ds(s * RPT + i * CHUNK, CHUNK)])
    plsc.subcore_barrier()

    def fire_gathers(g, buf):
        for k in range(K):
            pltpu.async_copy(tab_sh.at[src_v.at[g * K + k]],
                             rows_v.at[buf + k], gsem)

    def drain_gathers(g, buf):
        for k in range(K):
            pltpu.make_async_copy(tab_sh.at[src_v.at[g * K + k]],
                                  rows_v.at[buf + k], gsem).wait()

    def fire_scatters(g, buf):
        for k in range(K):
            pltpu.async_copy(rows_v.at[buf + k],
                             acc_sh.at[dsto_v.at[g * K + k]], ssem, add=True)

    def drain_scatters(g, buf):
        for k in range(K):
            pltpu.make_async_copy(rows_v.at[buf + k],
                                  acc_sh.at[dsto_v.at[g * K + k]], ssem).wait()

    fire_gathers(0, 0)

    def gbody(g, carry):
        buf = (g % 2) * K
        obuf = K - buf
        drain_gathers(g, buf)

        @pl.when(g >= 1)
        def _():
            drain_scatters(g - 1, obuf)

        @pl.when(g + 1 < G)
        def _():
            fire_gathers(g + 1, obuf)

        fire_scatters(g, buf)
        return carry

    lax.fori_loop(0, G, gbody, 0)
    drain_scatters(G - 1, ((G - 1) % 2) * K)
    plsc.subcore_barrier()
    for i in range(RPT // CHUNK):
        pltpu.sync_copy(acc_sh.at[pl.ds(s * RPT + i * CHUNK, CHUNK)],
                        rows_v.at[0])
        pltpu.sync_copy(rows_v.at[0],
                        s1_out.at[c, pl.ds(s * RPT + i * CHUNK, CHUNK)])


_sc_fused1 = functools.partial(
    pl.kernel,
    mesh=_MESH,
    compiler_params=_SC_PARAMS,
    out_type=[
        jax.ShapeDtypeStruct((2, NP, F1), jnp.float32),
        jax.ShapeDtypeStruct((2, NP), jnp.float32),
    ],
    scratch_types=[
        pltpu.VMEM((CPT, CHUNK), jnp.int32),
        pltpu.VMEM((CPT, CHUNK), jnp.int32),
        pltpu.VMEM((CPT, CHUNK), jnp.int32),
        pltpu.VMEM((CPT, CHUNK), jnp.int32),
        pltpu.VMEM((2 * K_BY_F[F1], CHUNK, F1), jnp.float32),
        pltpu.VMEM((CHUNK,), jnp.float32),
        pltpu.VMEM((RPT,), jnp.float32),
        pltpu.VMEM((RPT,), jnp.float32),
        pltpu.VMEM((CHUNK, F1), jnp.float32),
        pltpu.VMEM_SHARED((NP,), jnp.float32),
        pltpu.VMEM_SHARED((NP, F1), jnp.float32),
        pltpu.VMEM_SHARED((NP, F1), jnp.float32),
        pltpu.SemaphoreType.DMA,
        pltpu.SemaphoreType.DMA,
    ],
)(_fused1_body)


def _scat_body(F, src_hbm, dst_hbm, tab_hbm, out_hbm, src_v, dst_v, rows_v,
               acc_sh, gsem, ssem, *maybe_tab_sh):
    K = K_BY_F[F]
    G = CPT // K
    c = lax.axis_index("c")
    s = lax.axis_index("s")
    w = s * 2 + c
    pltpu.sync_copy(src_hbm.at[w], src_v)
    pltpu.sync_copy(dst_hbm.at[w], dst_v)
    if maybe_tab_sh:
        tab_src = maybe_tab_sh[0]
        pltpu.sync_copy(tab_hbm.at[pl.ds(s * RPT, RPT)],
                        tab_src.at[pl.ds(s * RPT, RPT)])
    else:
        tab_src = tab_hbm
    zero = jnp.zeros((16,), jnp.float32)

    def zrow(i, carry):
        for t in range(F // 16):
            rows_v[0, i, pl.ds(16 * t, 16)] = zero
        return carry

    lax.fori_loop(0, CHUNK, zrow, 0)
    for i in range(RPT // CHUNK):
        pltpu.sync_copy(rows_v.at[0],
                        acc_sh.at[pl.ds(s * RPT + i * CHUNK, CHUNK)])
    plsc.subcore_barrier()

    def fire_gathers(g, buf):
        for k in range(K):
            pltpu.async_copy(tab_src.at[src_v.at[g * K + k]],
                             rows_v.at[buf + k], gsem)

    def drain_gathers(g, buf):
        for k in range(K):
            pltpu.make_async_copy(tab_src.at[src_v.at[g * K + k]],
                                  rows_v.at[buf + k], gsem).wait()

    def fire_scatters(g, buf):
        for k in range(K):
            pltpu.async_copy(rows_v.at[buf + k],
                             acc_sh.at[dst_v.at[g * K + k]], ssem, add=True)

    def drain_scatters(g, buf):
        for k in range(K):
            pltpu.make_async_copy(rows_v.at[buf + k],
                                  acc_sh.at[dst_v.at[g * K + k]], ssem).wait()

    fire_gathers(0, 0)

    def gbody(g, carry):
        buf = (g % 2) * K
        obuf = K - buf
        drain_gathers(g, buf)

        @pl.when(g >= 1)
        def _():
            drain_scatters(g - 1, obuf)

        @pl.when(g + 1 < G)
        def _():
            fire_gathers(g + 1, obuf)

        fire_scatters(g, buf)
        return carry

    lax.fori_loop(0, G, gbody, 0)
    drain_scatters(G - 1, ((G - 1) % 2) * K)
    plsc.subcore_barrier()
    for i in range(RPT // CHUNK):
        pltpu.sync_copy(acc_sh.at[pl.ds(s * RPT + i * CHUNK, CHUNK)],
                        rows_v.at[0])
        pltpu.sync_copy(rows_v.at[0],
                        out_hbm.at[c, pl.ds(s * RPT + i * CHUNK, CHUNK)])


def _make_scat(F):
    scratch = [
        pltpu.VMEM((CPT, CHUNK), jnp.int32),
        pltpu.VMEM((CPT, CHUNK), jnp.int32),
        pltpu.VMEM((2 * K_BY_F[F], CHUNK, F), jnp.float32),
        pltpu.VMEM_SHARED((NP, F), jnp.float32),
        pltpu.SemaphoreType.DMA,
        pltpu.SemaphoreType.DMA,
    ]
    if SPMEM_TAB_BY_F[F]:
        scratch.append(pltpu.VMEM_SHARED((NP, F), jnp.float32))
    return functools.partial(
        pl.kernel,
        mesh=_MESH,
        compiler_params=_SC_PARAMS,
        out_type=jax.ShapeDtypeStruct((2, NP, F), jnp.float32),
        scratch_types=scratch,
    )(functools.partial(_scat_body, F))


_sc_scat48 = _make_scat(F2)


def _tc1_body(x_ref, w1_ref, h_ref):
    h_ref[...] = jnp.dot(x_ref[...], w1_ref[...],
                         preferred_element_type=jnp.float32)


def _tc1(x, W1):
    return pl.pallas_call(
        _tc1_body,
        grid=(N // BLK,),
        in_specs=[
            pl.BlockSpec((BLK, 128), lambda m: (m, 0)),
            pl.BlockSpec((128, F1), lambda m: (0, 0)),
        ],
        out_specs=pl.BlockSpec((BLK, F1), lambda m: (m, 0)),
        out_shape=jax.ShapeDtypeStruct((N, F1), jnp.float32),
    )(x, W1)


def _tc2_body(s1p_ref, h1_ref, dinv_ref, b1_ref, w2p_ref, hs2_ref):
    dinv = dinv_ref[...]
    hs1 = h1_ref[...] * dinv
    agg = dinv * (s1p_ref[0] + s1p_ref[1] + hs1) + b1_ref[...]
    g = jnp.maximum(agg, 0.0)
    h2 = jnp.dot(g, w2p_ref[...], preferred_element_type=jnp.float32)
    hs2_ref[...] = h2 * dinv


def _tc2(s1p, hs1, dinv, b1, W2p):
    return pl.pallas_call(
        _tc2_body,
        grid=(N // BLK,),
        in_specs=[
            pl.BlockSpec((2, BLK, F1), lambda m: (0, m, 0)),
            pl.BlockSpec((BLK, F1), lambda m: (m, 0)),
            pl.BlockSpec((BLK, 1), lambda m: (m, 0)),
            pl.BlockSpec((1, F1), lambda m: (0, 0)),
            pl.BlockSpec((F1, F2), lambda m: (0, 0)),
        ],
        out_specs=pl.BlockSpec((BLK, F2), lambda m: (m, 0)),
        out_shape=jax.ShapeDtypeStruct((N, F2), jnp.float32),
    )(s1p, hs1, dinv, b1, W2p)


def _tc3_body(s2p_ref, hs2_ref, dinv_ref, b2p_ref, out_ref):
    a = dinv_ref[...] * (s2p_ref[0] + s2p_ref[1] + hs2_ref[...]) + b2p_ref[...]
    a = a[:, :40]
    m = jnp.max(a, axis=1, keepdims=True)
    z = a - m
    lse = jnp.log(jnp.sum(jnp.exp(z), axis=1, keepdims=True))
    out_ref[...] = z - lse


def _tc3(s2p, hs2, dinv, b2p):
    return pl.pallas_call(
        _tc3_body,
        grid=(N // BLK,),
        in_specs=[
            pl.BlockSpec((2, BLK, F2), lambda m: (0, m, 0)),
            pl.BlockSpec((BLK, F2), lambda m: (m, 0)),
            pl.BlockSpec((BLK, 1), lambda m: (m, 0)),
            pl.BlockSpec((1, F2), lambda m: (0, 0)),
        ],
        out_specs=pl.BlockSpec((BLK, 40), lambda m: (m, 0)),
        out_shape=jax.ShapeDtypeStruct((N, 40), jnp.float32),
    )(s2p, hs2, dinv, b2p)


def kernel(x, edge_index, W1, b1, W2, b2):
    ei = edge_index.astype(jnp.int32)
    pad = EPAD - E
    src_p = jnp.concatenate([ei[0], jnp.zeros((pad,), jnp.int32)])
    dst_p = jnp.concatenate([ei[1], jnp.full((pad,), NP - 1, jnp.int32)])
    src_p = src_p.reshape(NW, CPT, CHUNK)
    dst_p = dst_p.reshape(NW, CPT, CHUNK)
    W2p = jnp.pad(W2, ((0, 0), (0, F2 - 40)))
    b2p = jnp.pad(b2, (0, F2 - 40)).reshape(1, F2)

    h1 = _tc1(x, W1)
    s1p, dinvp = _sc_fused1(src_p, dst_p, jnp.pad(h1, ((0, NP - N), (0, 0))))
    dinv = dinvp[0, :N, None]
    hs2 = _tc2(s1p, h1, dinv, b1.reshape(1, F1), W2p)
    s2p = _sc_scat48(src_p, dst_p, jnp.pad(hs2, ((0, NP - N), (0, 0))))
    return _tc3(s2p, hs2, dinv, b2p)


# speedup vs baseline: 1.9640x; 1.0385x over previous
import functools

import jax
import jax.numpy as jnp
from jax import lax
from jax.experimental import pallas as pl
from jax.experimental.pallas import tpu as pltpu
from jax.experimental.pallas import tpu_sc as plsc

N = 10000
E = 320000
NP = 10240
NW = 32
CHUNK = 128
CPT = 80
EPAD = NW * CPT * CHUNK
RPT = NP // 16
F1 = 16
F2 = 48
K_BY_F = {F1: 8, F2: 3}
SPMEM_TAB_BY_F = {F1: True, F2: True}
BLK = 1000

_MESH = plsc.VectorSubcoreMesh(core_axis_name="c", subcore_axis_name="s")
_SC_PARAMS = pltpu.CompilerParams(use_tc_tiling_on_sc=False,
                                  needs_layout_passes=False)


def _fused1_body(src_hbm, dst_hbm, tab_hbm, s1_out, dinv_out, src_v, dsta_v,
                 dstb_v, dsto_v, rows_v, ones_v, degb_v, dinv_v, tabb_v,
                 hist_sh, acc_sh, tab_sh, gsem, ssem):
    K = K_BY_F[F1]
    G = CPT // K
    c = lax.axis_index("c")
    s = lax.axis_index("s")
    w = s * 2 + c
    pltpu.sync_copy(src_hbm.at[w], src_v)
    pltpu.sync_copy(dst_hbm.at[w], dsto_v)
    pltpu.sync_copy(dst_hbm.at[s * 2], dsta_v)
    pltpu.sync_copy(dst_hbm.at[s * 2 + 1], dstb_v)
    pltpu.sync_copy(tab_hbm.at[pl.ds(s * RPT, RPT)],
                    tab_sh.at[pl.ds(s * RPT, RPT)])
    one = jnp.ones((16,), jnp.float32)
    zero = jnp.zeros((16,), jnp.float32)

    def obody(i, carry):
        ones_v[pl.ds(i * 16, 16)] = one
        return carry

    lax.fori_loop(0, CHUNK // 16, obody, 0)

    def zb(i, carry):
        degb_v[pl.ds(i * 16, 16)] = zero
        return carry

    lax.fori_loop(0, RPT // 16, zb, 0)
    pltpu.sync_copy(degb_v, hist_sh.at[pl.ds(s * RPT, RPT)])

    def zrow(i, carry):
        rows_v[0, i, :] = zero
        return carry

    lax.fori_loop(0, CHUNK, zrow, 0)
    for i in range(RPT // CHUNK):
        pltpu.sync_copy(rows_v.at[0],
                        acc_sh.at[pl.ds(s * RPT + i * CHUNK, CHUNK)])
    plsc.subcore_barrier()

    for dv in (dsta_v, dstb_v):
        def hb(g, carry, dv=dv):
            for k in range(16):
                pltpu.async_copy(ones_v, hist_sh.at[dv.at[g * 16 + k]], ssem,
                                 add=True)
            for k in range(16):
                pltpu.make_async_copy(ones_v, hist_sh.at[dv.at[g * 16 + k]],
                                      ssem).wait()
            return carry

        lax.fori_loop(0, CPT // 16, hb, 0)
    plsc.subcore_barrier()

    pltpu.sync_copy(hist_sh.at[pl.ds(s * RPT, RPT)], degb_v)
    half = jnp.full((16,), 0.5, jnp.float32)
    th = jnp.full((16,), 1.5, jnp.float32)

    def db(i, carry):
        d = degb_v[pl.ds(i * 16, 16)] + 1.0
        ii = plsc.bitcast(d, jnp.int32)
        ii = 0x5F3759DF - lax.shift_right_logical(ii, 1)
        y = plsc.bitcast(ii, jnp.float32)
        hx = d * half
        for _ in range(3):
            y = y * (th - hx * y * y)
        dinv_v[pl.ds(i * 16, 16)] = y
        return carry

    lax.fori_loop(0, RPT // 16, db, 0)
    pltpu.sync_copy(dinv_v, dinv_out.at[c, pl.ds(s * RPT, RPT)])
    for i in range(RPT // CHUNK):
        pltpu.sync_copy(tab_sh.at[pl.ds(s * RPT + i * CHUNK, CHUNK)], tabb_v)

        def sb(rb, carry, i=i):
            dv16 = dinv_v[pl.ds(i * CHUNK + rb * 16, 16)]
            for r in range(16):
                b = jnp.broadcast_to(dv16[r], (16,))
                tabb_v[rb * 16 + r, :] = tabb_v[rb * 16 + r, :] * b
            return carry

        lax.fori_loop(0, CHUNK // 16, sb, 0)
        pltpu.sync_copy(tabb_v, tab_sh.at[pl.ds(s * RPT + i * CHUNK, CHUNK)])
    plsc.subcore_barrier()

    def fire_gathers(g, buf):
        for k in range(K):
            pltpu.async_copy(tab_sh.at[src_v.at[g * K + k]],
                             rows_v.at[buf + k], gsem)

    def drain_gathers(g, buf):
        for k in range(K):
            pltpu.make_async_copy(tab_sh.at[src_v.at[g * K + k]],
                                  rows_v.at[buf + k], gsem).wait()

    def fire_scatters(g, buf):
        for k in range(K):
            pltpu.async_copy(rows_v.at[buf + k],
                             acc_sh.at[dsto_v.at[g * K + k]], ssem, add=True)

    def drain_scatters(g, buf):
        for k in range(K):
            pltpu.make_async_copy(rows_v.at[buf + k],
                                  acc_sh.at[dsto_v.at[g * K + k]], ssem).wait()

    fire_gathers(0, 0)

    def gbody(g, carry):
        buf = (g % 2) * K
        obuf = K - buf
        drain_gathers(g, buf)

        @pl.when(g >= 1)
        def _():
            drain_scatters(g - 1, obuf)

        @pl.when(g + 1 < G)
        def _():
            fire_gathers(g + 1, obuf)

        fire_scatters(g, buf)
        return carry

    lax.fori_loop(0, G, gbody, 0)
    drain_scatters(G - 1, ((G - 1) % 2) * K)
    plsc.subcore_barrier()
    for i in range(RPT // CHUNK):
        pltpu.sync_copy(acc_sh.at[pl.ds(s * RPT + i * CHUNK, CHUNK)],
                        rows_v.at[0])
        pltpu.sync_copy(rows_v.at[0],
                        s1_out.at[c, pl.ds(s * RPT + i * CHUNK, CHUNK)])


_sc_fused1 = functools.partial(
    pl.kernel,
    mesh=_MESH,
    compiler_params=_SC_PARAMS,
    out_type=[
        jax.ShapeDtypeStruct((2, NP, F1), jnp.float32),
        jax.ShapeDtypeStruct((2, NP), jnp.float32),
    ],
    scratch_types=[
        pltpu.VMEM((CPT, CHUNK), jnp.int32),
        pltpu.VMEM((CPT, CHUNK), jnp.int32),
        pltpu.VMEM((CPT, CHUNK), jnp.int32),
        pltpu.VMEM((CPT, CHUNK), jnp.int32),
        pltpu.VMEM((2 * K_BY_F[F1], CHUNK, F1), jnp.float32),
        pltpu.VMEM((CHUNK,), jnp.float32),
        pltpu.VMEM((RPT,), jnp.float32),
        pltpu.VMEM((RPT,), jnp.float32),
        pltpu.VMEM((CHUNK, F1), jnp.float32),
        pltpu.VMEM_SHARED((NP,), jnp.float32),
        pltpu.VMEM_SHARED((NP, F1), jnp.float32),
        pltpu.VMEM_SHARED((NP, F1), jnp.float32),
        pltpu.SemaphoreType.DMA,
        pltpu.SemaphoreType.DMA,
    ],
)(_fused1_body)


def _scat_body(F, src_hbm, dst_hbm, tab_hbm, out_hbm, src_v, dst_v, rows_v,
               acc_sh, gsem, ssem, *maybe_tab_sh):
    K = K_BY_F[F]
    G = CPT // K
    c = lax.axis_index("c")
    s = lax.axis_index("s")
    w = s * 2 + c
    pltpu.sync_copy(src_hbm.at[w], src_v)
    pltpu.sync_copy(dst_hbm.at[w], dst_v)
    if maybe_tab_sh:
        tab_src = maybe_tab_sh[0]
        pltpu.sync_copy(tab_hbm.at[pl.ds(s * RPT, RPT)],
                        tab_src.at[pl.ds(s * RPT, RPT)])
    else:
        tab_src = tab_hbm
    zero = jnp.zeros((16,), jnp.float32)

    def zrow(i, carry):
        for t in range(F // 16):
            rows_v[0, i, pl.ds(16 * t, 16)] = zero
        return carry

    lax.fori_loop(0, CHUNK, zrow, 0)
    for i in range(RPT // CHUNK):
        pltpu.sync_copy(rows_v.at[0],
                        acc_sh.at[pl.ds(s * RPT + i * CHUNK, CHUNK)])
    plsc.subcore_barrier()

    def fire_gathers(g, buf):
        for k in range(K):
            pltpu.async_copy(tab_src.at[src_v.at[g * K + k]],
                             rows_v.at[buf + k], gsem)

    def drain_gathers(g, buf):
        for k in range(K):
            pltpu.make_async_copy(tab_src.at[src_v.at[g * K + k]],
                                  rows_v.at[buf + k], gsem).wait()

    def fire_scatters(g, buf):
        for k in range(K):
            pltpu.async_copy(rows_v.at[buf + k],
                             acc_sh.at[dst_v.at[g * K + k]], ssem, add=True)

    def drain_scatters(g, buf):
        for k in range(K):
            pltpu.make_async_copy(rows_v.at[buf + k],
                                  acc_sh.at[dst_v.at[g * K + k]], ssem).wait()

    fire_gathers(0, 0)

    def gbody(g, carry):
        buf = (g % 2) * K
        obuf = K - buf
        drain_gathers(g, buf)

        @pl.when(g >= 1)
        def _():
            drain_scatters(g - 1, obuf)

        @pl.when(g + 1 < G)
        def _():
            fire_gathers(g + 1, obuf)

        fire_scatters(g, buf)
        return carry

    lax.fori_loop(0, G, gbody, 0)
    drain_scatters(G - 1, ((G - 1) % 2) * K)
    plsc.subcore_barrier()
    for i in range(RPT // CHUNK):
        pltpu.sync_copy(acc_sh.at[pl.ds(s * RPT + i * CHUNK, CHUNK)],
                        rows_v.at[0])
        pltpu.sync_copy(rows_v.at[0],
                        out_hbm.at[c, pl.ds(s * RPT + i * CHUNK, CHUNK)])


def _make_scat(F):
    scratch = [
        pltpu.VMEM((CPT, CHUNK), jnp.int32),
        pltpu.VMEM((CPT, CHUNK), jnp.int32),
        pltpu.VMEM((2 * K_BY_F[F], CHUNK, F), jnp.float32),
        pltpu.VMEM_SHARED((NP, F), jnp.float32),
        pltpu.SemaphoreType.DMA,
        pltpu.SemaphoreType.DMA,
    ]
    if SPMEM_TAB_BY_F[F]:
        scratch.append(pltpu.VMEM_SHARED((NP, F), jnp.float32))
    return functools.partial(
        pl.kernel,
        mesh=_MESH,
        compiler_params=_SC_PARAMS,
        out_type=jax.ShapeDtypeStruct((2, NP, F), jnp.float32),
        scratch_types=scratch,
    )(functools.partial(_scat_body, F))


_sc_scat48 = _make_scat(F2)


def _tc1_body(x_ref, w1_ref, h_ref):
    h_ref[...] = jnp.dot(x_ref[...], w1_ref[...],
                         preferred_element_type=jnp.float32)


def _tc1(x, W1):
    return pl.pallas_call(
        _tc1_body,
        grid=(N // BLK,),
        in_specs=[
            pl.BlockSpec((BLK, 128), lambda m: (m, 0)),
            pl.BlockSpec((128, F1), lambda m: (0, 0)),
        ],
        out_specs=pl.BlockSpec((BLK, F1), lambda m: (m, 0)),
        out_shape=jax.ShapeDtypeStruct((NP, F1), jnp.float32),
    )(x, W1)


def _tc2_body(s1p_ref, h1_ref, dinv_ref, b1_ref, w2p_ref, hs2_ref):
    dinv = dinv_ref[...]
    hs1 = h1_ref[...] * dinv
    agg = dinv * (s1p_ref[0] + s1p_ref[1] + hs1) + b1_ref[...]
    g = jnp.maximum(agg, 0.0)
    h2 = jnp.dot(g, w2p_ref[...], preferred_element_type=jnp.float32)
    hs2_ref[...] = h2 * dinv


def _tc2(s1p, hs1, dinv, b1, W2p):
    return pl.pallas_call(
        _tc2_body,
        grid=(N // BLK,),
        in_specs=[
            pl.BlockSpec((2, BLK, F1), lambda m: (0, m, 0)),
            pl.BlockSpec((BLK, F1), lambda m: (m, 0)),
            pl.BlockSpec((BLK, 1), lambda m: (m, 0)),
            pl.BlockSpec((1, F1), lambda m: (0, 0)),
            pl.BlockSpec((F1, F2), lambda m: (0, 0)),
        ],
        out_specs=pl.BlockSpec((BLK, F2), lambda m: (m, 0)),
        out_shape=jax.ShapeDtypeStruct((NP, F2), jnp.float32),
    )(s1p, hs1, dinv, b1, W2p)


def _tc3_body(s2p_ref, hs2_ref, dinv_ref, b2p_ref, out_ref):
    a = dinv_ref[...] * (s2p_ref[0] + s2p_ref[1] + hs2_ref[...]) + b2p_ref[...]
    a = a[:, :40]
    m = jnp.max(a, axis=1, keepdims=True)
    z = a - m
    lse = jnp.log(jnp.sum(jnp.exp(z), axis=1, keepdims=True))
    out_ref[...] = z - lse


def _tc3(s2p, hs2, dinv, b2p):
    return pl.pallas_call(
        _tc3_body,
        grid=(N // BLK,),
        in_specs=[
            pl.BlockSpec((2, BLK, F2), lambda m: (0, m, 0)),
            pl.BlockSpec((BLK, F2), lambda m: (m, 0)),
            pl.BlockSpec((BLK, 1), lambda m: (m, 0)),
            pl.BlockSpec((1, F2), lambda m: (0, 0)),
        ],
        out_specs=pl.BlockSpec((BLK, 40), lambda m: (m, 0)),
        out_shape=jax.ShapeDtypeStruct((N, 40), jnp.float32),
    )(s2p, hs2, dinv, b2p)


def kernel(x, edge_index, W1, b1, W2, b2):
    ei = edge_index.astype(jnp.int32)
    pad = EPAD - E
    src_p = jnp.concatenate([ei[0], jnp.zeros((pad,), jnp.int32)])
    dst_p = jnp.concatenate([ei[1], jnp.full((pad,), NP - 1, jnp.int32)])
    src_p = src_p.reshape(NW, CPT, CHUNK)
    dst_p = dst_p.reshape(NW, CPT, CHUNK)
    W2p = jnp.pad(W2, ((0, 0), (0, F2 - 40)))
    b2p = jnp.pad(b2, (0, F2 - 40)).reshape(1, F2)

    h1 = _tc1(x, W1)
    s1p, dinvp = _sc_fused1(src_p, dst_p, h1)
    dinv = dinvp[0, :N, None]
    hs2 = _tc2(s1p, h1, dinv, b1.reshape(1, F1), W2p)
    s2p = _sc_scat48(src_p, dst_p, hs2)
    return _tc3(s2p, hs2, dinv, b2p)
